# Initial kernel scaffold; baseline (speedup 1.0000x reference)
import functools

import jax
import jax.numpy as jnp
from jax import lax
from jax.experimental import pallas as pl
from jax.experimental.pallas import tpu as pltpu
from jax.experimental.pallas import tpu_sc as plsc

S = 4096
B = 128
BR = 512
BC = 512
NB = S // BR
SCALE_R = 64.0
SCALE_C = 128.0
MATCH = 128.0 * SCALE_R * SCALE_C
NEG = -3e9


def _match_body(x_ref, out_ref):
    i = pl.program_id(0)
    xr = x_ref[pl.ds(i * BR, BR), :]
    br = jnp.where(xr > 0.5, SCALE_R, -SCALE_R).astype(jnp.bfloat16)
    col_iota = lax.broadcasted_iota(jnp.float32, (BR, BC), 1)

    def dot_with_block(j):
        xc = x_ref[pl.ds(j * BC, BC), :]
        bcb = jnp.where(xc > 0.5, SCALE_C, -SCALE_C).astype(jnp.bfloat16)
        return lax.dot_general(br, bcb, (((1,), (1,)), ((), ())),
                               preferred_element_type=jnp.float32)

    def body(j, carry):
        score = dot_with_block(j) - (col_iota + (j * BC).astype(jnp.float32))
        return jnp.maximum(carry, jnp.max(score, axis=1))

    carry = lax.fori_loop(0, i, body, jnp.full((BR,), NEG, jnp.float32))

    row_iota = lax.broadcasted_iota(jnp.float32, (BR, BC), 0)
    col = col_iota + (i * BC).astype(jnp.float32)
    score = jnp.where(col_iota <= row_iota, dot_with_block(i) - col, NEG)
    carry = jnp.maximum(carry, jnp.max(score, axis=1))
    out_ref[0, 0, :] = (MATCH - carry).astype(jnp.int32)


def _best_indices(x, interpret=False):
    out = pl.pallas_call(
        _match_body,
        grid=(NB,),
        in_specs=[pl.BlockSpec((S, B), lambda i: (0, 0))],
        out_specs=pl.BlockSpec((1, 1, BR), lambda i: (i, 0, 0)),
        out_shape=jax.ShapeDtypeStruct((NB, 1, BR), jnp.int32),
        interpret=interpret,
    )(x)
    return out.reshape(S)


_NW = 32
_BPW = S // _NW


def _sc_gather(x, idx):
    mesh = plsc.VectorSubcoreMesh(core_axis_name="c", subcore_axis_name="s")

    @functools.partial(
        pl.kernel,
        out_type=jax.ShapeDtypeStruct((S, B), jnp.float32),
        mesh=mesh,
        scratch_types=[
            pltpu.VMEM((_BPW,), jnp.int32),
            pltpu.VMEM((_BPW, B), jnp.float32),
            pltpu.SemaphoreType.DMA,
        ],
    )
    def k(table_hbm, idx_hbm, out_hbm, idx_v, rows_v, sem):
        wid = lax.axis_index("s") * 2 + lax.axis_index("c")
        base = wid * _BPW
        pltpu.sync_copy(idx_hbm.at[pl.ds(base, _BPW)], idx_v)
        pltpu.async_copy(table_hbm.at[idx_v], rows_v, sem).wait()
        pltpu.sync_copy(rows_v, out_hbm.at[pl.ds(base, _BPW)])

    return k(x, idx)


def kernel(x):
    best = _best_indices(x)
    return _sc_gather(x, best)


# speedup vs baseline: 1.6527x; 1.6527x over previous
import functools

import jax
import jax.numpy as jnp
from jax import lax
from jax.experimental import pallas as pl
from jax.experimental.pallas import tpu as pltpu
from jax.experimental.pallas import tpu_sc as plsc

S = 4096
B = 128
BR = 512
BC = 512
NB = S // BR
SCALE_R = 64.0
SCALE_C = 128.0
MATCH = 128.0 * SCALE_R * SCALE_C
NEG = -3e9


def _match_body(x_ref, out_ref):
    i = pl.program_id(0)
    xr = x_ref[pl.ds(i * BR, BR), :]
    br = jnp.where(xr > 0.5, SCALE_R, -SCALE_R).astype(jnp.bfloat16)
    col_iota = lax.broadcasted_iota(jnp.int32, (BR, BC), 1).astype(jnp.float32)

    def dot_with_block(j):
        xc = x_ref[pl.ds(j * BC, BC), :]
        bcb = jnp.where(xc > 0.5, SCALE_C, -SCALE_C).astype(jnp.bfloat16)
        return lax.dot_general(br, bcb, (((1,), (1,)), ((), ())),
                               preferred_element_type=jnp.float32)

    def body(j, carry):
        score = dot_with_block(j) - (col_iota + (j * BC).astype(jnp.float32))
        return jnp.maximum(carry, jnp.max(score, axis=1))

    carry = lax.fori_loop(0, i, body, jnp.full((BR,), NEG, jnp.float32))

    row_iota = lax.broadcasted_iota(jnp.int32, (BR, BC), 0).astype(jnp.float32)
    col = col_iota + (i * BC).astype(jnp.float32)
    score = jnp.where(col_iota <= row_iota, dot_with_block(i) - col, NEG)
    carry = jnp.maximum(carry, jnp.max(score, axis=1))
    out_ref[0, 0, :] = (MATCH - carry).astype(jnp.int32)


def _best_indices(x, interpret=False):
    out = pl.pallas_call(
        _match_body,
        grid=(NB,),
        in_specs=[pl.BlockSpec((S, B), lambda i: (0, 0))],
        out_specs=pl.BlockSpec((1, 1, BR), lambda i: (i, 0, 0)),
        out_shape=jax.ShapeDtypeStruct((NB, 1, BR), jnp.int32),
        interpret=interpret,
    )(x)
    return out.reshape(S)


_NW = 32
_BPW = S // _NW


def _sc_gather(x, idx):
    mesh = plsc.VectorSubcoreMesh(core_axis_name="c", subcore_axis_name="s")

    @functools.partial(
        pl.kernel,
        out_type=jax.ShapeDtypeStruct((S, B), jnp.float32),
        mesh=mesh,
        scratch_types=[
            pltpu.VMEM((_BPW,), jnp.int32),
            pltpu.VMEM((_BPW, B), jnp.float32),
            pltpu.SemaphoreType.DMA,
        ],
    )
    def k(table_hbm, idx_hbm, out_hbm, idx_v, rows_v, sem):
        wid = lax.axis_index("s") * 2 + lax.axis_index("c")
        base = wid * _BPW
        pltpu.sync_copy(idx_hbm.at[pl.ds(base, _BPW)], idx_v)
        pltpu.async_copy(table_hbm.at[idx_v], rows_v, sem).wait()
        pltpu.sync_copy(rows_v, out_hbm.at[pl.ds(base, _BPW)])

    return k(x, idx)


def kernel(x):
    best = _best_indices(x)
    return _sc_gather(x, best)


# speedup vs baseline: 1.7492x; 1.0584x over previous
import functools

import jax
import jax.numpy as jnp
from jax import lax
from jax.experimental import pallas as pl
from jax.experimental.pallas import tpu as pltpu
from jax.experimental.pallas import tpu_sc as plsc

S = 4096
B = 128
BR = 512
BC = 512
NB = S // BR
NQ = BC // 128
SCALE_R = 64.0
SCALE_C = 128.0
MATCH = 128.0 * SCALE_R * SCALE_C
NEG = -3e9


def _match_body(x_ref, out_ref, sgn_r_ref, sgn_c_ref):
    i = pl.program_id(0)

    @pl.when(i == 0)
    def _precompute():
        xv = x_ref[...]
        m = xv > 0.5
        sgn_r_ref[...] = jnp.where(m, SCALE_R, -SCALE_R).astype(jnp.bfloat16)
        sgn_c_ref[...] = jnp.where(m, SCALE_C, -SCALE_C).astype(jnp.bfloat16)

    br = sgn_r_ref[pl.ds(i * BR, BR), :]

    def dot_with_block(j):
        bcb = sgn_c_ref[pl.ds(j * BC, BC), :]
        return lax.dot_general(br, bcb, (((1,), (1,)), ((), ())),
                               preferred_element_type=jnp.float32)

    def body(j, carry):
        d = dot_with_block(j)
        base = (j * BC).astype(jnp.float32)
        for q in range(NQ):
            dq = d[:, q * 128:(q + 1) * 128] - (base + 128.0 * q)
            carry = jnp.maximum(carry, dq)
        return carry

    carry = lax.fori_loop(0, i, body,
                          jnp.full((BR, 128), NEG, jnp.float32))

    d = dot_with_block(i)
    r_iota = lax.broadcasted_iota(jnp.int32, (BR, 128), 0)
    lane = lax.broadcasted_iota(jnp.int32, (BR, 128), 1)
    base = (i * BC).astype(jnp.float32)
    for q in range(NQ):
        cond = (lane + q * 128) <= r_iota
        dq = d[:, q * 128:(q + 1) * 128] - (base + 128.0 * q)
        carry = jnp.maximum(carry, jnp.where(cond, dq, NEG))

    score = jnp.max(carry - lane.astype(jnp.float32), axis=1)
    out_ref[0, 0, :] = (MATCH - score).astype(jnp.int32)


def _best_indices(x, interpret=False):
    out = pl.pallas_call(
        _match_body,
        grid=(NB,),
        in_specs=[pl.BlockSpec((S, B), lambda i: (0, 0))],
        out_specs=pl.BlockSpec((1, 1, BR), lambda i: (i, 0, 0)),
        out_shape=jax.ShapeDtypeStruct((NB, 1, BR), jnp.int32),
        scratch_shapes=[
            pltpu.VMEM((S, B), jnp.bfloat16),
            pltpu.VMEM((S, B), jnp.bfloat16),
        ],
        interpret=interpret,
    )(x)
    return out.reshape(S)


_NW = 32
_BPW = S // _NW


def _sc_gather(x, idx):
    mesh = plsc.VectorSubcoreMesh(core_axis_name="c", subcore_axis_name="s")

    @functools.partial(
        pl.kernel,
        out_type=jax.ShapeDtypeStruct((S, B), jnp.float32),
        mesh=mesh,
        scratch_types=[
            pltpu.VMEM((_BPW,), jnp.int32),
            pltpu.VMEM((_BPW, B), jnp.float32),
            pltpu.SemaphoreType.DMA,
        ],
    )
    def k(table_hbm, idx_hbm, out_hbm, idx_v, rows_v, sem):
        wid = lax.axis_index("s") * 2 + lax.axis_index("c")
        base = wid * _BPW
        pltpu.sync_copy(idx_hbm.at[pl.ds(base, _BPW)], idx_v)
        pltpu.async_copy(table_hbm.at[idx_v], rows_v, sem).wait()
        pltpu.sync_copy(rows_v, out_hbm.at[pl.ds(base, _BPW)])

    return k(x, idx)


def kernel(x):
    best = _best_indices(x)
    return _sc_gather(x, best)


# speedup vs baseline: 1.9373x; 1.1075x over previous
import functools

import jax
import jax.numpy as jnp
from jax import lax
from jax.experimental import pallas as pl
from jax.experimental.pallas import tpu as pltpu
from jax.experimental.pallas import tpu_sc as plsc

S = 4096
B = 128
BR = 1024
BC = 512
NB = S // BR
NQ = BC // 128
SCALE_R = 64.0
SCALE_C = 128.0
MATCH = 128.0 * SCALE_R * SCALE_C
NEG = -3e9


def _match_body(x_ref, out_ref, sgn_r_ref, sgn_c_ref):
    i = pl.program_id(0)

    @pl.when(i == 0)
    def _precompute():
        xv = x_ref[...]
        m = xv > 0.5
        sgn_r_ref[...] = jnp.where(m, SCALE_R, -SCALE_R).astype(jnp.bfloat16)
        sgn_c_ref[...] = jnp.where(m, SCALE_C, -SCALE_C).astype(jnp.bfloat16)

    br = sgn_r_ref[pl.ds(i * BR, BR), :]

    def dot_with_block(j):
        bcb = sgn_c_ref[pl.ds(j * BC, BC), :]
        return lax.dot_general(br, bcb, (((1,), (1,)), ((), ())),
                               preferred_element_type=jnp.float32)

    def body(j, carry):
        d = dot_with_block(j)
        base = (j * BC).astype(jnp.float32)
        for q in range(NQ):
            dq = d[:, q * 128:(q + 1) * 128] - (base + 128.0 * q)
            carry = jnp.maximum(carry, dq)
        return carry

    RBC = BR // BC
    carry = lax.fori_loop(0, i * RBC, body,
                          jnp.full((BR, 128), NEG, jnp.float32))

    r_iota = lax.broadcasted_iota(jnp.int32, (BR, 128), 0)
    lane = lax.broadcasted_iota(jnp.int32, (BR, 128), 1)
    for k in range(RBC):
        d = dot_with_block(i * RBC + k)
        base = ((i * RBC + k) * BC).astype(jnp.float32)
        for q in range(NQ):
            cond = (lane + (k * BC + q * 128)) <= r_iota
            dq = d[:, q * 128:(q + 1) * 128] - (base + 128.0 * q)
            carry = jnp.maximum(carry, jnp.where(cond, dq, NEG))

    score = jnp.max(carry - lane.astype(jnp.float32), axis=1)
    out_ref[0, 0, :] = (MATCH - score).astype(jnp.int32)


def _best_indices(x, interpret=False):
    out = pl.pallas_call(
        _match_body,
        grid=(NB,),
        in_specs=[pl.BlockSpec((S, B), lambda i: (0, 0))],
        out_specs=pl.BlockSpec((1, 1, BR), lambda i: (i, 0, 0)),
        out_shape=jax.ShapeDtypeStruct((NB, 1, BR), jnp.int32),
        scratch_shapes=[
            pltpu.VMEM((S, B), jnp.bfloat16),
            pltpu.VMEM((S, B), jnp.bfloat16),
        ],
        interpret=interpret,
    )(x)
    return out.reshape(S)


_NW = 32
_BPW = S // _NW


def _sc_gather(x, idx):
    mesh = plsc.VectorSubcoreMesh(core_axis_name="c", subcore_axis_name="s")

    @functools.partial(
        pl.kernel,
        out_type=jax.ShapeDtypeStruct((S, B), jnp.float32),
        mesh=mesh,
        scratch_types=[
            pltpu.VMEM((_BPW,), jnp.int32),
            pltpu.VMEM((_BPW, B), jnp.float32),
            pltpu.SemaphoreType.DMA,
        ],
    )
    def k(table_hbm, idx_hbm, out_hbm, idx_v, rows_v, sem):
        wid = lax.axis_index("s") * 2 + lax.axis_index("c")
        base = wid * _BPW
        pltpu.sync_copy(idx_hbm.at[pl.ds(base, _BPW)], idx_v)
        pltpu.async_copy(table_hbm.at[idx_v], rows_v, sem).wait()
        pltpu.sync_copy(rows_v, out_hbm.at[pl.ds(base, _BPW)])

    return k(x, idx)


def kernel(x):
    best = _best_indices(x)
    return _sc_gather(x, best)
